# baseline (speedup 1.0000x reference)
import jax
import jax.numpy as jnp
from jax import lax
from jax.experimental import pallas as pl
from jax.experimental.pallas import tpu as pltpu, tpu_sc as plsc
import functools

D_MODEL = 1024
BATCH = 4
SEQ = 4096
CHUNK = 16

_info = plsc.get_sparse_core_info()
NC, NS, LANES = _info.num_cores, _info.num_subcores, _info.num_lanes
NW = NC * NS
SEQ_PER_W = SEQ // NW
N_CHUNKS = SEQ_PER_W // CHUNK


def _body(x_hbm, pos_hbm, out_hbm, pos_buf, x_buf, sem):
    cid = lax.axis_index("c")
    sid = lax.axis_index("s")
    wid = sid * NC + cid
    seq0 = wid * SEQ_PER_W

    def chunk_step(ci, _):
        s0 = seq0 + ci * CHUNK
        pltpu.sync_copy(pos_hbm.at[pl.ds(s0, CHUNK), :], pos_buf)
        for b in range(BATCH):
            pltpu.sync_copy(x_hbm.at[b, pl.ds(s0, CHUNK), :], x_buf)

            def row_step(r, _):
                for c in range(D_MODEL // LANES):
                    v = pos_buf[r, pl.ds(c * LANES, LANES)]
                    plsc.addupdate(x_buf.at[r, pl.ds(c * LANES, LANES)], v)
                return 0

            lax.fori_loop(0, CHUNK, row_step, 0)
            pltpu.sync_copy(x_buf, out_hbm.at[b, pl.ds(s0, CHUNK), :])
        return 0

    lax.fori_loop(0, N_CHUNKS, chunk_step, 0)


@jax.jit
def kernel(x, pos_table):
    mesh = plsc.VectorSubcoreMesh(core_axis_name="c", subcore_axis_name="s")
    return pl.kernel(
        _body,
        out_type=jax.ShapeDtypeStruct((BATCH, SEQ, D_MODEL), jnp.float32),
        mesh=mesh,
        scratch_types=[
            pltpu.VMEM((CHUNK, D_MODEL), jnp.float32),
            pltpu.VMEM((CHUNK, D_MODEL), jnp.float32),
            pltpu.SemaphoreType.DMA,
        ],
    )(x, pos_table)
